# Initial kernel scaffold; baseline (speedup 1.0000x reference)
import jax
import jax.numpy as jnp
from jax.experimental import pallas as pl


def kernel(x, edge_index, edge_weight, W0, b0, W1, b1, W2, b2, Wm1, bm1, Wm2, bm2):
    raise NotImplementedError("write your pallas kernel here")


# speedup vs baseline: 15.5465x; 15.5465x over previous
import functools

import jax
import jax.numpy as jnp
from jax import lax
from jax.experimental import pallas as pl
from jax.experimental.pallas import tpu as pltpu
from jax.experimental.pallas import tpu_sc as plsc

N = 10000
E = 320000
D = 128

NC = 2
NS = 16
NW = NC * NS

C = 128
CH = 80
EPW = C * CH
EP = EPW * NW

NP = 10240
RPT = NP // NS

BR = 1000
GRID = N // BR


def _bcast16(vec, l):
    dnums = lax.GatherDimensionNumbers(
        offset_dims=(), collapsed_slice_dims=(0,), start_index_map=(0,))
    idx = jnp.full((16, 1), l, dtype=jnp.int32)
    return lax.gather(vec, idx, dnums, slice_sizes=(1,),
                      mode=lax.GatherScatterMode.PROMISE_IN_BOUNDS)


def _sc_degree(dst3, ew3):
    mesh = plsc.VectorSubcoreMesh(core_axis_name="c", subcore_axis_name="s")

    @functools.partial(
        pl.kernel,
        out_type=jax.ShapeDtypeStruct((NC, NP), jnp.float32),
        mesh=mesh,
        scratch_types=[
            pltpu.VMEM((CH, C), jnp.int32),
            pltpu.VMEM((CH, C), jnp.float32),
            pltpu.VMEM((RPT,), jnp.float32),
            pltpu.VMEM_SHARED((NP,), jnp.float32),
        ],
    )
    def deg_kernel(dst_hbm, ew_hbm, out_hbm, dst_v, ew_v, zb, acc):
        cid = lax.axis_index("c")
        sid = lax.axis_index("s")
        wid = cid * NS + sid

        zero = jnp.zeros((16,), jnp.float32)

        def zr(i, carry):
            zb[pl.ds(i * 16, 16)] = zero
            return carry
        lax.fori_loop(0, RPT // 16, zr, 0)

        pltpu.sync_copy(dst_hbm.at[wid], dst_v)
        pltpu.sync_copy(ew_hbm.at[wid], ew_v)

        pltpu.sync_copy(zb, acc.at[pl.ds(sid * RPT, RPT)])
        plsc.subcore_barrier()

        def chunk(t, carry):
            pltpu.sync_copy(ew_v.at[t], acc.at[dst_v.at[t]], add=True)
            return carry
        lax.fori_loop(0, CH, chunk, 0)

        plsc.subcore_barrier()
        pltpu.sync_copy(acc.at[pl.ds(sid * RPT, RPT)],
                        out_hbm.at[cid, pl.ds(sid * RPT, RPT)])

    return deg_kernel(dst3, ew3)


def _sc_agg(u, src3, dst3, ew3):
    mesh = plsc.VectorSubcoreMesh(core_axis_name="c", subcore_axis_name="s")

    @functools.partial(
        pl.kernel,
        out_type=jax.ShapeDtypeStruct((NC, NP, D), jnp.float32),
        mesh=mesh,
        scratch_types=[
            pltpu.VMEM((CH, C), jnp.int32),
            pltpu.VMEM((CH, C), jnp.int32),
            pltpu.VMEM((CH, C), jnp.float32),
            pltpu.VMEM((C, D), jnp.float32),
            pltpu.VMEM_SHARED((NP, D), jnp.float32),
            pltpu.SemaphoreType.DMA,
        ],
    )
    def agg_kernel(u_hbm, src_hbm, dst_hbm, ew_hbm, out_hbm,
                   src_v, dst_v, ew_v, rows_v, acc, sem):
        cid = lax.axis_index("c")
        sid = lax.axis_index("s")
        wid = cid * NS + sid

        zero = jnp.zeros((16,), jnp.float32)

        def zr(r, carry):
            for j in range(D // 16):
                rows_v[r, pl.ds(j * 16, 16)] = zero
            return carry
        lax.fori_loop(0, C, zr, 0)

        for k in range(RPT // C):
            pltpu.sync_copy(rows_v, acc.at[pl.ds(sid * RPT + k * C, C)])

        pltpu.sync_copy(src_hbm.at[wid], src_v)
        pltpu.sync_copy(dst_hbm.at[wid], dst_v)
        pltpu.sync_copy(ew_hbm.at[wid], ew_v)
        plsc.subcore_barrier()

        def chunk(t, carry):
            pltpu.async_copy(u_hbm.at[src_v.at[t]], rows_v, sem).wait()

            def scale(g, carry2):
                wv = ew_v[t, pl.ds(g * 16, 16)]
                for l in range(16):
                    w = _bcast16(wv, l)
                    r = g * 16 + l
                    for j in range(D // 16):
                        rows_v[r, pl.ds(j * 16, 16)] = (
                            rows_v[r, pl.ds(j * 16, 16)] * w)
                return carry2
            lax.fori_loop(0, C // 16, scale, 0)

            pltpu.sync_copy(rows_v, acc.at[dst_v.at[t]], add=True)
            return carry
        lax.fori_loop(0, CH, chunk, 0)

        plsc.subcore_barrier()
        pltpu.sync_copy(acc.at[pl.ds(sid * RPT, RPT)],
                        out_hbm.at[cid, pl.ds(sid * RPT, RPT)])

    return agg_kernel(u, src3, dst3, ew3)


def _tc_first(degT, x, W0):
    def body(deg_ref, x_ref, w_ref, u_ref, dinv_ref):
        d = deg_ref[...]
        dinv = lax.rsqrt(1.0 + d[:, 0:1] + d[:, 1:2])
        dinv_ref[...] = dinv
        u_ref[...] = jnp.dot(x_ref[...] * dinv, w_ref[...],
                             preferred_element_type=jnp.float32)

    return pl.pallas_call(
        body,
        grid=(GRID,),
        in_specs=[
            pl.BlockSpec((BR, 2), lambda i: (i, 0)),
            pl.BlockSpec((BR, D), lambda i: (i, 0)),
            pl.BlockSpec((D, D), lambda i: (0, 0)),
        ],
        out_specs=[
            pl.BlockSpec((BR, D), lambda i: (i, 0)),
            pl.BlockSpec((BR, 1), lambda i: (i, 0)),
        ],
        out_shape=[
            jax.ShapeDtypeStruct((N, D), jnp.float32),
            jax.ShapeDtypeStruct((N, 1), jnp.float32),
        ],
    )(degT, x, W0)


def _tc_mid(sp, u, dinv, b, W):
    def body(sp_ref, u_ref, dinv_ref, b_ref, w_ref, out_ref):
        dinv = dinv_ref[...]
        s = sp_ref[0] + sp_ref[1] + u_ref[...]
        h = jnp.maximum(dinv * s + b_ref[...], 0.0)
        out_ref[...] = jnp.dot(dinv * h, w_ref[...],
                               preferred_element_type=jnp.float32)

    return pl.pallas_call(
        body,
        grid=(GRID,),
        in_specs=[
            pl.BlockSpec((2, BR, D), lambda i: (0, i, 0)),
            pl.BlockSpec((BR, D), lambda i: (i, 0)),
            pl.BlockSpec((BR, 1), lambda i: (i, 0)),
            pl.BlockSpec((1, D), lambda i: (0, 0)),
            pl.BlockSpec((D, D), lambda i: (0, 0)),
        ],
        out_specs=pl.BlockSpec((BR, D), lambda i: (i, 0)),
        out_shape=jax.ShapeDtypeStruct((N, D), jnp.float32),
    )(sp, u, dinv, b, W)


def _tc_head(sp, u, dinv, b2, Wm1, bm1, Wm2, bm2):
    H = Wm1.shape[1]
    L = Wm2.shape[1]

    def body(sp_ref, u_ref, dinv_ref, b2_ref, wm1_ref, bm1_ref,
             wm2_ref, bm2_ref, out_ref):
        dinv = dinv_ref[...]
        s = sp_ref[0] + sp_ref[1] + u_ref[...]
        h3 = jnp.maximum(dinv * s + b2_ref[...], 0.0)
        hm = jnp.maximum(
            jnp.dot(h3, wm1_ref[...], preferred_element_type=jnp.float32)
            + bm1_ref[...], 0.0)
        out_ref[...] = (
            jnp.dot(hm, wm2_ref[...], preferred_element_type=jnp.float32)
            + bm2_ref[...])

    return pl.pallas_call(
        body,
        grid=(GRID,),
        in_specs=[
            pl.BlockSpec((2, BR, D), lambda i: (0, i, 0)),
            pl.BlockSpec((BR, D), lambda i: (i, 0)),
            pl.BlockSpec((BR, 1), lambda i: (i, 0)),
            pl.BlockSpec((1, D), lambda i: (0, 0)),
            pl.BlockSpec((D, H), lambda i: (0, 0)),
            pl.BlockSpec((1, H), lambda i: (0, 0)),
            pl.BlockSpec((H, L), lambda i: (0, 0)),
            pl.BlockSpec((1, L), lambda i: (0, 0)),
        ],
        out_specs=pl.BlockSpec((BR, L), lambda i: (i, 0)),
        out_shape=jax.ShapeDtypeStruct((N, L), jnp.float32),
    )(sp, u, dinv, b2, Wm1, bm1, Wm2, bm2)


def kernel(x, edge_index, edge_weight, W0, b0, W1, b1, W2, b2,
           Wm1, bm1, Wm2, bm2):
    src = edge_index[0].astype(jnp.int32)
    dst = edge_index[1].astype(jnp.int32)
    ew = edge_weight.astype(jnp.float32)

    pad = EP - E
    fill = (jnp.arange(pad, dtype=jnp.int32) * 13) % N
    src3 = jnp.concatenate([src, fill]).reshape(NW, CH, C)
    dst3 = jnp.concatenate([dst, fill]).reshape(NW, CH, C)
    ew3 = jnp.concatenate([ew, jnp.zeros((pad,), jnp.float32)]
                          ).reshape(NW, CH, C)

    degp = _sc_degree(dst3, ew3)
    degT = degp[:, :N].T

    u0, dinv = _tc_first(degT, x, W0)
    sp0 = _sc_agg(u0, src3, dst3, ew3)
    u1 = _tc_mid(sp0, u0, dinv, b0.reshape(1, D), W1)
    sp1 = _sc_agg(u1, src3, dst3, ew3)
    u2 = _tc_mid(sp1, u1, dinv, b1.reshape(1, D), W2)
    sp2 = _sc_agg(u2, src3, dst3, ew3)
    out = _tc_head(sp2, u2, dinv, b2.reshape(1, D),
                   Wm1, bm1.reshape(1, -1), Wm2, bm2.reshape(1, -1))
    return out


# speedup vs baseline: 25.4326x; 1.6359x over previous
import functools

import jax
import jax.numpy as jnp
from jax import lax
from jax.experimental import pallas as pl
from jax.experimental.pallas import tpu as pltpu
from jax.experimental.pallas import tpu_sc as plsc

N = 10000
E = 320000
D = 128

NC = 2
NS = 16
NW = NC * NS

C = 80
CH = 128
EPW = C * CH
EP = EPW * NW

NP = 10240
RPT = NP // NS

BR = 1000
GRID = N // BR


def _bcast16(vec, l):
    dnums = lax.GatherDimensionNumbers(
        offset_dims=(), collapsed_slice_dims=(0,), start_index_map=(0,))
    idx = jnp.full((16, 1), l, dtype=jnp.int32)
    return lax.gather(vec, idx, dnums, slice_sizes=(1,),
                      mode=lax.GatherScatterMode.PROMISE_IN_BOUNDS)


def _sc_degree(dst3, ew3):
    mesh = plsc.VectorSubcoreMesh(core_axis_name="c", subcore_axis_name="s")

    @functools.partial(
        pl.kernel,
        out_type=jax.ShapeDtypeStruct((NC, NP), jnp.float32),
        mesh=mesh,
        scratch_types=[
            pltpu.VMEM((CH, C), jnp.int32),
            pltpu.VMEM((CH, C), jnp.float32),
            pltpu.VMEM((RPT,), jnp.float32),
            pltpu.VMEM_SHARED((NP,), jnp.float32),
        ],
    )
    def deg_kernel(dst_hbm, ew_hbm, out_hbm, dst_v, ew_v, zb, acc):
        cid = lax.axis_index("c")
        sid = lax.axis_index("s")
        wid = cid * NS + sid

        zero = jnp.zeros((16,), jnp.float32)

        def zr(i, carry):
            zb[pl.ds(i * 16, 16)] = zero
            return carry
        lax.fori_loop(0, RPT // 16, zr, 0)

        pltpu.sync_copy(dst_hbm.at[wid], dst_v)
        pltpu.sync_copy(ew_hbm.at[wid], ew_v)

        pltpu.sync_copy(zb, acc.at[pl.ds(sid * RPT, RPT)])
        plsc.subcore_barrier()

        def chunk(t, carry):
            pltpu.sync_copy(ew_v.at[t], acc.at[dst_v.at[t]], add=True)
            return carry
        lax.fori_loop(0, CH, chunk, 0)

        plsc.subcore_barrier()
        pltpu.sync_copy(acc.at[pl.ds(sid * RPT, RPT)],
                        out_hbm.at[cid, pl.ds(sid * RPT, RPT)])

    return deg_kernel(dst3, ew3)


NB = 3


def _sc_agg(u, src3, dst3, ew3):
    mesh = plsc.VectorSubcoreMesh(core_axis_name="c", subcore_axis_name="s")

    @functools.partial(
        pl.kernel,
        out_type=jax.ShapeDtypeStruct((NC, NP, D), jnp.float32),
        mesh=mesh,
        scratch_types=[
            pltpu.VMEM((CH, C), jnp.int32),
            pltpu.VMEM((NB, C), jnp.int32),
            pltpu.VMEM((NB, C), jnp.float32),
            pltpu.VMEM((NB, C, D), jnp.float32),
            pltpu.VMEM_SHARED((NP, D), jnp.float32),
            [pltpu.SemaphoreType.DMA] * NB,
            [pltpu.SemaphoreType.DMA] * NB,
            [pltpu.SemaphoreType.DMA] * NB,
        ],
    )
    def agg_kernel(u_hbm, src_hbm, dst_hbm, ew_hbm, out_hbm,
                   src_v, dst_r, ew_r, rows_v, acc, gsem, isem, ssem):
        cid = lax.axis_index("c")
        sid = lax.axis_index("s")
        wid = cid * NS + sid

        zero = jnp.zeros((16,), jnp.float32)

        def zr(r, carry):
            for j in range(D // 16):
                rows_v[0, r, pl.ds(j * 16, 16)] = zero
            return carry
        lax.fori_loop(0, C, zr, 0)

        for k in range(RPT // C):
            pltpu.sync_copy(rows_v.at[0], acc.at[pl.ds(sid * RPT + k * C, C)])

        pltpu.sync_copy(src_hbm.at[wid], src_v)
        plsc.subcore_barrier()

        def issue(g, slot, drain):
            if drain:
                pltpu.make_async_copy(rows_v.at[slot], acc.at[dst_r.at[0]],
                                      ssem[slot]).wait()
            pltpu.async_copy(u_hbm.at[src_v.at[g]], rows_v.at[slot],
                             gsem[slot])
            pltpu.async_copy(dst_hbm.at[wid, g], dst_r.at[slot], isem[slot])
            pltpu.async_copy(ew_hbm.at[wid, g], ew_r.at[slot], isem[slot])

        def consume(t, p):
            pltpu.make_async_copy(u_hbm.at[src_v.at[0]], rows_v.at[p],
                                  gsem[p]).wait()
            pltpu.make_async_copy(dst_hbm.at[wid, 0], dst_r.at[p],
                                  isem[p]).wait()
            pltpu.make_async_copy(ew_hbm.at[wid, 0], ew_r.at[p],
                                  isem[p]).wait()

            def scale(q, carry2):
                wv = ew_r[p, pl.ds(q * 16, 16)]
                for l in range(16):
                    w = _bcast16(wv, l)
                    r = q * 16 + l
                    for j in range(D // 16):
                        rows_v[p, r, pl.ds(j * 16, 16)] = (
                            rows_v[p, r, pl.ds(j * 16, 16)] * w)
                return carry2
            lax.fori_loop(0, C // 16, scale, 0)

            pltpu.async_copy(rows_v.at[p], acc.at[dst_r.at[p]], ssem[p],
                             add=True)

        issue(0, 0, drain=False)
        issue(1, 1, drain=False)

        NMAIN = CH - CH % NB

        def outer(o, carry):
            for p in range(NB):
                t = o * NB + p
                g = t + 2
                bg = (p + 2) % NB

                @pl.when(g < CH)
                def _issue():
                    @pl.when(g >= NB)
                    def _dr():
                        pltpu.make_async_copy(rows_v.at[bg],
                                              acc.at[dst_r.at[0]],
                                              ssem[bg]).wait()
                    pltpu.async_copy(u_hbm.at[src_v.at[g]], rows_v.at[bg],
                                     gsem[bg])
                    pltpu.async_copy(dst_hbm.at[wid, g], dst_r.at[bg],
                                     isem[bg])
                    pltpu.async_copy(ew_hbm.at[wid, g], ew_r.at[bg],
                                     isem[bg])

                consume(t, p)
            return carry
        lax.fori_loop(0, NMAIN // NB, outer, 0)

        for t in range(NMAIN, CH):
            consume(t, t % NB)

        for b in range(NB):
            pltpu.make_async_copy(rows_v.at[b], acc.at[dst_r.at[0]],
                                  ssem[b]).wait()

        plsc.subcore_barrier()
        pltpu.sync_copy(acc.at[pl.ds(sid * RPT, RPT)],
                        out_hbm.at[cid, pl.ds(sid * RPT, RPT)])

    return agg_kernel(u, src3, dst3, ew3)


def _tc_first(degT, x, W0):
    def body(deg_ref, x_ref, w_ref, u_ref, dinv_ref):
        d = deg_ref[...]
        dinv = lax.rsqrt(1.0 + d[:, 0:1] + d[:, 1:2])
        dinv_ref[...] = dinv
        u_ref[...] = jnp.dot(x_ref[...] * dinv, w_ref[...],
                             preferred_element_type=jnp.float32)

    return pl.pallas_call(
        body,
        grid=(GRID,),
        in_specs=[
            pl.BlockSpec((BR, 2), lambda i: (i, 0)),
            pl.BlockSpec((BR, D), lambda i: (i, 0)),
            pl.BlockSpec((D, D), lambda i: (0, 0)),
        ],
        out_specs=[
            pl.BlockSpec((BR, D), lambda i: (i, 0)),
            pl.BlockSpec((BR, 1), lambda i: (i, 0)),
        ],
        out_shape=[
            jax.ShapeDtypeStruct((N, D), jnp.float32),
            jax.ShapeDtypeStruct((N, 1), jnp.float32),
        ],
    )(degT, x, W0)


def _tc_mid(sp, u, dinv, b, W):
    def body(sp_ref, u_ref, dinv_ref, b_ref, w_ref, out_ref):
        dinv = dinv_ref[...]
        s = sp_ref[0] + sp_ref[1] + u_ref[...]
        h = jnp.maximum(dinv * s + b_ref[...], 0.0)
        out_ref[...] = jnp.dot(dinv * h, w_ref[...],
                               preferred_element_type=jnp.float32)

    return pl.pallas_call(
        body,
        grid=(GRID,),
        in_specs=[
            pl.BlockSpec((2, BR, D), lambda i: (0, i, 0)),
            pl.BlockSpec((BR, D), lambda i: (i, 0)),
            pl.BlockSpec((BR, 1), lambda i: (i, 0)),
            pl.BlockSpec((1, D), lambda i: (0, 0)),
            pl.BlockSpec((D, D), lambda i: (0, 0)),
        ],
        out_specs=pl.BlockSpec((BR, D), lambda i: (i, 0)),
        out_shape=jax.ShapeDtypeStruct((N, D), jnp.float32),
    )(sp, u, dinv, b, W)


def _tc_head(sp, u, dinv, b2, Wm1, bm1, Wm2, bm2):
    H = Wm1.shape[1]
    L = Wm2.shape[1]

    def body(sp_ref, u_ref, dinv_ref, b2_ref, wm1_ref, bm1_ref,
             wm2_ref, bm2_ref, out_ref):
        dinv = dinv_ref[...]
        s = sp_ref[0] + sp_ref[1] + u_ref[...]
        h3 = jnp.maximum(dinv * s + b2_ref[...], 0.0)
        hm = jnp.maximum(
            jnp.dot(h3, wm1_ref[...], preferred_element_type=jnp.float32)
            + bm1_ref[...], 0.0)
        out_ref[...] = (
            jnp.dot(hm, wm2_ref[...], preferred_element_type=jnp.float32)
            + bm2_ref[...])

    return pl.pallas_call(
        body,
        grid=(GRID,),
        in_specs=[
            pl.BlockSpec((2, BR, D), lambda i: (0, i, 0)),
            pl.BlockSpec((BR, D), lambda i: (i, 0)),
            pl.BlockSpec((BR, 1), lambda i: (i, 0)),
            pl.BlockSpec((1, D), lambda i: (0, 0)),
            pl.BlockSpec((D, H), lambda i: (0, 0)),
            pl.BlockSpec((1, H), lambda i: (0, 0)),
            pl.BlockSpec((H, L), lambda i: (0, 0)),
            pl.BlockSpec((1, L), lambda i: (0, 0)),
        ],
        out_specs=pl.BlockSpec((BR, L), lambda i: (i, 0)),
        out_shape=jax.ShapeDtypeStruct((N, L), jnp.float32),
    )(sp, u, dinv, b2, Wm1, bm1, Wm2, bm2)


def kernel(x, edge_index, edge_weight, W0, b0, W1, b1, W2, b2,
           Wm1, bm1, Wm2, bm2):
    src = edge_index[0].astype(jnp.int32)
    dst = edge_index[1].astype(jnp.int32)
    ew = edge_weight.astype(jnp.float32)

    pad = EP - E
    fill = (jnp.arange(pad, dtype=jnp.int32) * 13) % N
    src3 = jnp.concatenate([src, fill]).reshape(NW, CH, C)
    dst3 = jnp.concatenate([dst, fill]).reshape(NW, CH, C)
    ew3 = jnp.concatenate([ew, jnp.zeros((pad,), jnp.float32)]
                          ).reshape(NW, CH, C)

    degp = _sc_degree(dst3, ew3)
    degT = degp[:, :N].T

    u0, dinv = _tc_first(degT, x, W0)
    sp0 = _sc_agg(u0, src3, dst3, ew3)
    u1 = _tc_mid(sp0, u0, dinv, b0.reshape(1, D), W1)
    sp1 = _sc_agg(u1, src3, dst3, ew3)
    u2 = _tc_mid(sp1, u1, dinv, b1.reshape(1, D), W2)
    sp2 = _sc_agg(u2, src3, dst3, ew3)
    out = _tc_head(sp2, u2, dinv, b2.reshape(1, D),
                   Wm1, bm1.reshape(1, -1), Wm2, bm2.reshape(1, -1))
    return out


# speedup vs baseline: 26.2191x; 1.0309x over previous
import functools

import jax
import jax.numpy as jnp
from jax import lax
from jax.experimental import pallas as pl
from jax.experimental.pallas import tpu as pltpu
from jax.experimental.pallas import tpu_sc as plsc

N = 10000
E = 320000
D = 128

NC = 2
NS = 16
NW = NC * NS

C = 128
CH = 80
EPW = C * CH
EP = EPW * NW

NP = 10240
RPT = NP // NS
RAT = 624

BR = 1000
GRID = N // BR


def _bcast16(vec, l):
    dnums = lax.GatherDimensionNumbers(
        offset_dims=(), collapsed_slice_dims=(0,), start_index_map=(0,))
    idx = jnp.full((16, 1), l, dtype=jnp.int32)
    return lax.gather(vec, idx, dnums, slice_sizes=(1,),
                      mode=lax.GatherScatterMode.PROMISE_IN_BOUNDS)


def _sc_degree(dst3, ew3):
    mesh = plsc.VectorSubcoreMesh(core_axis_name="c", subcore_axis_name="s")

    @functools.partial(
        pl.kernel,
        out_type=jax.ShapeDtypeStruct((NC, NP), jnp.float32),
        mesh=mesh,
        scratch_types=[
            pltpu.VMEM((CH, C), jnp.int32),
            pltpu.VMEM((CH, C), jnp.float32),
            pltpu.VMEM((RPT,), jnp.float32),
            pltpu.VMEM_SHARED((NP,), jnp.float32),
        ],
    )
    def deg_kernel(dst_hbm, ew_hbm, out_hbm, dst_v, ew_v, zb, acc):
        cid = lax.axis_index("c")
        sid = lax.axis_index("s")
        wid = cid * NS + sid

        zero = jnp.zeros((16,), jnp.float32)

        def zr(i, carry):
            zb[pl.ds(i * 16, 16)] = zero
            return carry
        lax.fori_loop(0, RPT // 16, zr, 0)

        pltpu.sync_copy(dst_hbm.at[wid], dst_v)
        pltpu.sync_copy(ew_hbm.at[wid], ew_v)

        pltpu.sync_copy(zb, acc.at[pl.ds(sid * RPT, RPT)])
        plsc.subcore_barrier()

        def chunk(t, carry):
            pltpu.sync_copy(ew_v.at[t], acc.at[dst_v.at[t]], add=True)
            return carry
        lax.fori_loop(0, CH, chunk, 0)

        plsc.subcore_barrier()
        pltpu.sync_copy(acc.at[pl.ds(sid * RPT, RPT)],
                        out_hbm.at[cid, pl.ds(sid * RPT, RPT)])

    return deg_kernel(dst3, ew3)


NB = 3
SRN = 3


def _sc_agg(u, src3, dst3, ew3):
    mesh = plsc.VectorSubcoreMesh(core_axis_name="c", subcore_axis_name="s")

    @functools.partial(
        pl.kernel,
        out_type=jax.ShapeDtypeStruct((NC, N, D), jnp.float32),
        mesh=mesh,
        scratch_types=[
            pltpu.VMEM((SRN, C), jnp.int32),
            pltpu.VMEM((NB, C), jnp.int32),
            pltpu.VMEM((NB, C), jnp.float32),
            pltpu.VMEM((NB, C, D), jnp.float32),
            pltpu.VMEM_SHARED((N, D), jnp.float32),
            [pltpu.SemaphoreType.DMA] * SRN,
            [pltpu.SemaphoreType.DMA] * NB,
            [pltpu.SemaphoreType.DMA] * NB,
            [pltpu.SemaphoreType.DMA] * NB,
        ],
    )
    def agg_kernel(u_hbm, src_hbm, dst_hbm, ew_hbm, out_hbm,
                   src_r, dst_r, ew_r, rows_v, acc, csem, gsem, isem, ssem):
        cid = lax.axis_index("c")
        sid = lax.axis_index("s")
        wid = cid * NS + sid

        zero = jnp.zeros((16,), jnp.float32)

        def zr(r, carry):
            for j in range(D // 16):
                rows_v[0, r, pl.ds(j * 16, 16)] = zero
            return carry
        lax.fori_loop(0, C, zr, 0)

        for k in range(RAT // C):
            pltpu.sync_copy(rows_v.at[0], acc.at[pl.ds(sid * RAT + k * C, C)])
        pltpu.sync_copy(rows_v.at[0, pl.ds(0, RAT % C)],
                        acc.at[pl.ds(sid * RAT + (RAT // C) * C, RAT % C)])

        @pl.when(sid == NS - 1)
        def _ztail():
            pltpu.sync_copy(rows_v.at[0, pl.ds(0, N - RAT * NS)],
                            acc.at[pl.ds(RAT * NS, N - RAT * NS)])

        def fetch_src(c, slot):
            pltpu.async_copy(src_hbm.at[wid, c], src_r.at[slot], csem[slot])

        def issue(g, slot, srcslot, drain):
            if drain:
                pltpu.make_async_copy(rows_v.at[slot], acc.at[dst_r.at[0]],
                                      ssem[slot]).wait()
            pltpu.make_async_copy(src_hbm.at[wid, 0], src_r.at[srcslot],
                                  csem[srcslot]).wait()
            pltpu.async_copy(u_hbm.at[src_r.at[srcslot]], rows_v.at[slot],
                             gsem[slot])
            pltpu.async_copy(dst_hbm.at[wid, g], dst_r.at[slot], isem[slot])
            pltpu.async_copy(ew_hbm.at[wid, g], ew_r.at[slot], isem[slot])

        def consume(t, p):
            pltpu.make_async_copy(u_hbm.at[src_r.at[0]], rows_v.at[p],
                                  gsem[p]).wait()
            pltpu.make_async_copy(dst_hbm.at[wid, 0], dst_r.at[p],
                                  isem[p]).wait()
            pltpu.make_async_copy(ew_hbm.at[wid, 0], ew_r.at[p],
                                  isem[p]).wait()

            def scale(q, carry2):
                wv = ew_r[p, pl.ds(q * 16, 16)]
                for l in range(16):
                    w = _bcast16(wv, l)
                    r = q * 16 + l
                    for j in range(D // 16):
                        rows_v[p, r, pl.ds(j * 16, 16)] = (
                            rows_v[p, r, pl.ds(j * 16, 16)] * w)
                return carry2
            lax.fori_loop(0, C // 16, scale, 0)

            pltpu.async_copy(rows_v.at[p], acc.at[dst_r.at[p]], ssem[p],
                             add=True)

        fetch_src(0, 0)
        fetch_src(1, 1)
        plsc.subcore_barrier()
        issue(0, 0, 0, drain=False)
        fetch_src(2, 2)
        issue(1, 1, 1, drain=False)
        fetch_src(3, 0)

        NMAIN = CH - CH % NB

        def outer(o, carry):
            for p in range(NB):
                t = o * NB + p
                g = t + 2
                bg = (p + 2) % NB
                sg = (p + 2) % SRN

                @pl.when(g < CH)
                def _issue():
                    @pl.when(g >= NB)
                    def _dr():
                        pltpu.make_async_copy(rows_v.at[bg],
                                              acc.at[dst_r.at[0]],
                                              ssem[bg]).wait()
                    pltpu.make_async_copy(src_hbm.at[wid, 0],
                                          src_r.at[sg], csem[sg]).wait()
                    pltpu.async_copy(u_hbm.at[src_r.at[sg]], rows_v.at[bg],
                                     gsem[bg])
                    pltpu.async_copy(dst_hbm.at[wid, g], dst_r.at[bg],
                                     isem[bg])
                    pltpu.async_copy(ew_hbm.at[wid, g], ew_r.at[bg],
                                     isem[bg])

                    @pl.when(g + 2 < CH)
                    def _pf():
                        pltpu.async_copy(src_hbm.at[wid, g + 2],
                                         src_r.at[(p + 1) % SRN],
                                         csem[(p + 1) % SRN])

                consume(t, p)
            return carry
        lax.fori_loop(0, NMAIN // NB, outer, 0)

        for t in range(NMAIN, CH):
            consume(t, t % NB)

        for b in range(NB):
            pltpu.make_async_copy(rows_v.at[b], acc.at[dst_r.at[0]],
                                  ssem[b]).wait()

        plsc.subcore_barrier()
        pltpu.sync_copy(acc.at[pl.ds(sid * RAT, RAT)],
                        out_hbm.at[cid, pl.ds(sid * RAT, RAT)])

        @pl.when(sid == NS - 1)
        def _otail():
            pltpu.sync_copy(acc.at[pl.ds(RAT * NS, N - RAT * NS)],
                            out_hbm.at[cid, pl.ds(RAT * NS, N - RAT * NS)])

    return agg_kernel(u, src3, dst3, ew3)


def _tc_first(degT, x, W0):
    def body(deg_ref, x_ref, w_ref, u_ref, dinv_ref):
        d = deg_ref[...]
        dinv = lax.rsqrt(1.0 + d[:, 0:1] + d[:, 1:2])
        dinv_ref[...] = dinv
        u_ref[...] = jnp.dot(x_ref[...] * dinv, w_ref[...],
                             preferred_element_type=jnp.float32)

    return pl.pallas_call(
        body,
        grid=(GRID,),
        in_specs=[
            pl.BlockSpec((BR, 2), lambda i: (i, 0)),
            pl.BlockSpec((BR, D), lambda i: (i, 0)),
            pl.BlockSpec((D, D), lambda i: (0, 0)),
        ],
        out_specs=[
            pl.BlockSpec((BR, D), lambda i: (i, 0)),
            pl.BlockSpec((BR, 1), lambda i: (i, 0)),
        ],
        out_shape=[
            jax.ShapeDtypeStruct((N, D), jnp.float32),
            jax.ShapeDtypeStruct((N, 1), jnp.float32),
        ],
    )(degT, x, W0)


def _tc_mid(sp, u, dinv, b, W):
    def body(sp_ref, u_ref, dinv_ref, b_ref, w_ref, out_ref):
        dinv = dinv_ref[...]
        s = sp_ref[0] + sp_ref[1] + u_ref[...]
        h = jnp.maximum(dinv * s + b_ref[...], 0.0)
        out_ref[...] = jnp.dot(dinv * h, w_ref[...],
                               preferred_element_type=jnp.float32)

    return pl.pallas_call(
        body,
        grid=(GRID,),
        in_specs=[
            pl.BlockSpec((2, BR, D), lambda i: (0, i, 0)),
            pl.BlockSpec((BR, D), lambda i: (i, 0)),
            pl.BlockSpec((BR, 1), lambda i: (i, 0)),
            pl.BlockSpec((1, D), lambda i: (0, 0)),
            pl.BlockSpec((D, D), lambda i: (0, 0)),
        ],
        out_specs=pl.BlockSpec((BR, D), lambda i: (i, 0)),
        out_shape=jax.ShapeDtypeStruct((N, D), jnp.float32),
    )(sp, u, dinv, b, W)


def _tc_head(sp, u, dinv, b2, Wm1, bm1, Wm2, bm2):
    H = Wm1.shape[1]
    L = Wm2.shape[1]

    def body(sp_ref, u_ref, dinv_ref, b2_ref, wm1_ref, bm1_ref,
             wm2_ref, bm2_ref, out_ref):
        dinv = dinv_ref[...]
        s = sp_ref[0] + sp_ref[1] + u_ref[...]
        h3 = jnp.maximum(dinv * s + b2_ref[...], 0.0)
        hm = jnp.maximum(
            jnp.dot(h3, wm1_ref[...], preferred_element_type=jnp.float32)
            + bm1_ref[...], 0.0)
        out_ref[...] = (
            jnp.dot(hm, wm2_ref[...], preferred_element_type=jnp.float32)
            + bm2_ref[...])

    return pl.pallas_call(
        body,
        grid=(GRID,),
        in_specs=[
            pl.BlockSpec((2, BR, D), lambda i: (0, i, 0)),
            pl.BlockSpec((BR, D), lambda i: (i, 0)),
            pl.BlockSpec((BR, 1), lambda i: (i, 0)),
            pl.BlockSpec((1, D), lambda i: (0, 0)),
            pl.BlockSpec((D, H), lambda i: (0, 0)),
            pl.BlockSpec((1, H), lambda i: (0, 0)),
            pl.BlockSpec((H, L), lambda i: (0, 0)),
            pl.BlockSpec((1, L), lambda i: (0, 0)),
        ],
        out_specs=pl.BlockSpec((BR, L), lambda i: (i, 0)),
        out_shape=jax.ShapeDtypeStruct((N, L), jnp.float32),
    )(sp, u, dinv, b2, Wm1, bm1, Wm2, bm2)


def kernel(x, edge_index, edge_weight, W0, b0, W1, b1, W2, b2,
           Wm1, bm1, Wm2, bm2):
    src = edge_index[0].astype(jnp.int32)
    dst = edge_index[1].astype(jnp.int32)
    ew = edge_weight.astype(jnp.float32)

    pad = EP - E
    fill = (jnp.arange(pad, dtype=jnp.int32) * 13) % N
    src3 = jnp.concatenate([src, fill]).reshape(NW, CH, C)
    dst3 = jnp.concatenate([dst, fill]).reshape(NW, CH, C)
    ew3 = jnp.concatenate([ew, jnp.zeros((pad,), jnp.float32)]
                          ).reshape(NW, CH, C)

    degp = _sc_degree(dst3, ew3)
    degT = degp[:, :N].T

    u0, dinv = _tc_first(degT, x, W0)
    sp0 = _sc_agg(u0, src3, dst3, ew3)
    u1 = _tc_mid(sp0, u0, dinv, b0.reshape(1, D), W1)
    sp1 = _sc_agg(u1, src3, dst3, ew3)
    u2 = _tc_mid(sp1, u1, dinv, b1.reshape(1, D), W2)
    sp2 = _sc_agg(u2, src3, dst3, ew3)
    out = _tc_head(sp2, u2, dinv, b2.reshape(1, D),
                   Wm1, bm1.reshape(1, -1), Wm2, bm2.reshape(1, -1))
    return out


# speedup vs baseline: 26.4194x; 1.0076x over previous
import functools

import jax
import jax.numpy as jnp
from jax import lax
from jax.experimental import pallas as pl
from jax.experimental.pallas import tpu as pltpu
from jax.experimental.pallas import tpu_sc as plsc

N = 10000
E = 320000
D = 128

NC = 2
NS = 16
NW = NC * NS

C = 80
CH = 128
EPW = C * CH
EP = EPW * NW

NP = 10240
RPT = NP // NS
RAT = 624

BR = 1000
GRID = N // BR


def _bcast16(vec, l):
    dnums = lax.GatherDimensionNumbers(
        offset_dims=(), collapsed_slice_dims=(0,), start_index_map=(0,))
    idx = jnp.full((16, 1), l, dtype=jnp.int32)
    return lax.gather(vec, idx, dnums, slice_sizes=(1,),
                      mode=lax.GatherScatterMode.PROMISE_IN_BOUNDS)


def _sc_degree(dst3, ew3):
    mesh = plsc.VectorSubcoreMesh(core_axis_name="c", subcore_axis_name="s")

    @functools.partial(
        pl.kernel,
        out_type=jax.ShapeDtypeStruct((NC, NP), jnp.float32),
        mesh=mesh,
        scratch_types=[
            pltpu.VMEM((CH, C), jnp.int32),
            pltpu.VMEM((CH, C), jnp.float32),
            pltpu.VMEM((RPT,), jnp.float32),
            pltpu.VMEM_SHARED((NP,), jnp.float32),
        ],
    )
    def deg_kernel(dst_hbm, ew_hbm, out_hbm, dst_v, ew_v, zb, acc):
        cid = lax.axis_index("c")
        sid = lax.axis_index("s")
        wid = cid * NS + sid

        zero = jnp.zeros((16,), jnp.float32)

        def zr(i, carry):
            zb[pl.ds(i * 16, 16)] = zero
            return carry
        lax.fori_loop(0, RPT // 16, zr, 0)

        pltpu.sync_copy(dst_hbm.at[wid], dst_v)
        pltpu.sync_copy(ew_hbm.at[wid], ew_v)

        pltpu.sync_copy(zb, acc.at[pl.ds(sid * RPT, RPT)])
        plsc.subcore_barrier()

        def chunk(t, carry):
            pltpu.sync_copy(ew_v.at[t], acc.at[dst_v.at[t]], add=True)
            return carry
        lax.fori_loop(0, CH, chunk, 0)

        plsc.subcore_barrier()
        pltpu.sync_copy(acc.at[pl.ds(sid * RPT, RPT)],
                        out_hbm.at[cid, pl.ds(sid * RPT, RPT)])

    return deg_kernel(dst3, ew3)


NB = 4
SRN = 8
PH = 8


def _sc_agg(u, src3, dst3, ew3):
    mesh = plsc.VectorSubcoreMesh(core_axis_name="c", subcore_axis_name="s")

    @functools.partial(
        pl.kernel,
        out_type=jax.ShapeDtypeStruct((NC, N, D), jnp.float32),
        mesh=mesh,
        scratch_types=[
            pltpu.VMEM((SRN, C), jnp.int32),
            pltpu.VMEM((NB, C), jnp.int32),
            pltpu.VMEM((NB, C), jnp.float32),
            pltpu.VMEM((NB, C, D), jnp.float32),
            pltpu.VMEM_SHARED((N, D), jnp.float32),
            [pltpu.SemaphoreType.DMA] * SRN,
            [pltpu.SemaphoreType.DMA] * NB,
            [pltpu.SemaphoreType.DMA] * NB,
            [pltpu.SemaphoreType.DMA] * NB,
        ],
    )
    def agg_kernel(u_hbm, src_hbm, dst_hbm, ew_hbm, out_hbm,
                   src_r, dst_r, ew_r, rows_v, acc, csem, gsem, isem, ssem):
        cid = lax.axis_index("c")
        sid = lax.axis_index("s")
        wid = cid * NS + sid

        zero = jnp.zeros((16,), jnp.float32)

        def zr(r, carry):
            for j in range(D // 16):
                rows_v[0, r, pl.ds(j * 16, 16)] = zero
            return carry
        lax.fori_loop(0, C, zr, 0)

        for k in range(RAT // C):
            pltpu.sync_copy(rows_v.at[0], acc.at[pl.ds(sid * RAT + k * C, C)])
        pltpu.sync_copy(rows_v.at[0, pl.ds(0, RAT % C)],
                        acc.at[pl.ds(sid * RAT + (RAT // C) * C, RAT % C)])

        @pl.when(sid == NS - 1)
        def _ztail():
            pltpu.sync_copy(rows_v.at[0, pl.ds(0, N - RAT * NS)],
                            acc.at[pl.ds(RAT * NS, N - RAT * NS)])

        def fetch_src(c, slot):
            pltpu.async_copy(src_hbm.at[wid, c], src_r.at[slot], csem[slot])

        def issue(g, slot, sslot, drain):
            if drain:
                pltpu.make_async_copy(rows_v.at[slot], acc.at[dst_r.at[0]],
                                      ssem[slot]).wait()
            pltpu.make_async_copy(src_hbm.at[wid, 0], src_r.at[sslot],
                                  csem[sslot]).wait()
            pltpu.async_copy(u_hbm.at[src_r.at[sslot]], rows_v.at[slot],
                             gsem[slot])
            pltpu.async_copy(dst_hbm.at[wid, g], dst_r.at[slot], isem[slot])
            pltpu.async_copy(ew_hbm.at[wid, g], ew_r.at[slot], isem[slot])

        def consume(t, p):
            pltpu.make_async_copy(u_hbm.at[src_r.at[0]], rows_v.at[p],
                                  gsem[p]).wait()
            pltpu.make_async_copy(dst_hbm.at[wid, 0], dst_r.at[p],
                                  isem[p]).wait()
            pltpu.make_async_copy(ew_hbm.at[wid, 0], ew_r.at[p],
                                  isem[p]).wait()

            def scale(q, carry2):
                wv = ew_r[p, pl.ds(q * 16, 16)]
                for l in range(16):
                    w = _bcast16(wv, l)
                    r = q * 16 + l
                    for j in range(D // 16):
                        rows_v[p, r, pl.ds(j * 16, 16)] = (
                            rows_v[p, r, pl.ds(j * 16, 16)] * w)
                return carry2
            lax.fori_loop(0, C // 16, scale, 0)

            pltpu.async_copy(rows_v.at[p], acc.at[dst_r.at[p]], ssem[p],
                             add=True)

        for c in range(6):
            fetch_src(c, c)
        plsc.subcore_barrier()
        issue(0, 0, 0, drain=False)
        issue(1, 1, 1, drain=False)

        def outer(o, carry):
            for p in range(PH):
                t = o * PH + p
                g = t + 2
                bg = (p + 2) % NB
                sg = (p + 2) % SRN

                @pl.when(g < CH)
                def _issue():
                    @pl.when(g >= NB)
                    def _dr():
                        pltpu.make_async_copy(rows_v.at[bg],
                                              acc.at[dst_r.at[0]],
                                              ssem[bg]).wait()
                    pltpu.make_async_copy(src_hbm.at[wid, 0],
                                          src_r.at[sg], csem[sg]).wait()
                    pltpu.async_copy(u_hbm.at[src_r.at[sg]], rows_v.at[bg],
                                     gsem[bg])
                    pltpu.async_copy(dst_hbm.at[wid, g], dst_r.at[bg],
                                     isem[bg])
                    pltpu.async_copy(ew_hbm.at[wid, g], ew_r.at[bg],
                                     isem[bg])

                    @pl.when(g + 4 < CH)
                    def _pf():
                        fetch_src(g + 4, (p + 6) % SRN)

                consume(t, p % NB)
            return carry
        lax.fori_loop(0, CH // PH, outer, 0)

        for b in range(NB):
            pltpu.make_async_copy(rows_v.at[b], acc.at[dst_r.at[0]],
                                  ssem[b]).wait()

        plsc.subcore_barrier()
        pltpu.sync_copy(acc.at[pl.ds(sid * RAT, RAT)],
                        out_hbm.at[cid, pl.ds(sid * RAT, RAT)])

        @pl.when(sid == NS - 1)
        def _otail():
            pltpu.sync_copy(acc.at[pl.ds(RAT * NS, N - RAT * NS)],
                            out_hbm.at[cid, pl.ds(RAT * NS, N - RAT * NS)])

    return agg_kernel(u, src3, dst3, ew3)


def _tc_first(degT, x, W0):
    def body(deg_ref, x_ref, w_ref, u_ref, dinv_ref):
        d = deg_ref[...]
        dinv = lax.rsqrt(1.0 + d[:, 0:1] + d[:, 1:2])
        dinv_ref[...] = dinv
        u_ref[...] = jnp.dot(x_ref[...] * dinv, w_ref[...],
                             preferred_element_type=jnp.float32)

    return pl.pallas_call(
        body,
        grid=(GRID,),
        in_specs=[
            pl.BlockSpec((BR, 2), lambda i: (i, 0)),
            pl.BlockSpec((BR, D), lambda i: (i, 0)),
            pl.BlockSpec((D, D), lambda i: (0, 0)),
        ],
        out_specs=[
            pl.BlockSpec((BR, D), lambda i: (i, 0)),
            pl.BlockSpec((BR, 1), lambda i: (i, 0)),
        ],
        out_shape=[
            jax.ShapeDtypeStruct((N, D), jnp.float32),
            jax.ShapeDtypeStruct((N, 1), jnp.float32),
        ],
    )(degT, x, W0)


def _tc_mid(sp, u, dinv, b, W):
    def body(sp_ref, u_ref, dinv_ref, b_ref, w_ref, out_ref):
        dinv = dinv_ref[...]
        s = sp_ref[0] + sp_ref[1] + u_ref[...]
        h = jnp.maximum(dinv * s + b_ref[...], 0.0)
        out_ref[...] = jnp.dot(dinv * h, w_ref[...],
                               preferred_element_type=jnp.float32)

    return pl.pallas_call(
        body,
        grid=(GRID,),
        in_specs=[
            pl.BlockSpec((2, BR, D), lambda i: (0, i, 0)),
            pl.BlockSpec((BR, D), lambda i: (i, 0)),
            pl.BlockSpec((BR, 1), lambda i: (i, 0)),
            pl.BlockSpec((1, D), lambda i: (0, 0)),
            pl.BlockSpec((D, D), lambda i: (0, 0)),
        ],
        out_specs=pl.BlockSpec((BR, D), lambda i: (i, 0)),
        out_shape=jax.ShapeDtypeStruct((N, D), jnp.float32),
    )(sp, u, dinv, b, W)


def _tc_head(sp, u, dinv, b2, Wm1, bm1, Wm2, bm2):
    H = Wm1.shape[1]
    L = Wm2.shape[1]

    def body(sp_ref, u_ref, dinv_ref, b2_ref, wm1_ref, bm1_ref,
             wm2_ref, bm2_ref, out_ref):
        dinv = dinv_ref[...]
        s = sp_ref[0] + sp_ref[1] + u_ref[...]
        h3 = jnp.maximum(dinv * s + b2_ref[...], 0.0)
        hm = jnp.maximum(
            jnp.dot(h3, wm1_ref[...], preferred_element_type=jnp.float32)
            + bm1_ref[...], 0.0)
        out_ref[...] = (
            jnp.dot(hm, wm2_ref[...], preferred_element_type=jnp.float32)
            + bm2_ref[...])

    return pl.pallas_call(
        body,
        grid=(GRID,),
        in_specs=[
            pl.BlockSpec((2, BR, D), lambda i: (0, i, 0)),
            pl.BlockSpec((BR, D), lambda i: (i, 0)),
            pl.BlockSpec((BR, 1), lambda i: (i, 0)),
            pl.BlockSpec((1, D), lambda i: (0, 0)),
            pl.BlockSpec((D, H), lambda i: (0, 0)),
            pl.BlockSpec((1, H), lambda i: (0, 0)),
            pl.BlockSpec((H, L), lambda i: (0, 0)),
            pl.BlockSpec((1, L), lambda i: (0, 0)),
        ],
        out_specs=pl.BlockSpec((BR, L), lambda i: (i, 0)),
        out_shape=jax.ShapeDtypeStruct((N, L), jnp.float32),
    )(sp, u, dinv, b2, Wm1, bm1, Wm2, bm2)


def kernel(x, edge_index, edge_weight, W0, b0, W1, b1, W2, b2,
           Wm1, bm1, Wm2, bm2):
    src = edge_index[0].astype(jnp.int32)
    dst = edge_index[1].astype(jnp.int32)
    ew = edge_weight.astype(jnp.float32)

    pad = EP - E
    fill = (jnp.arange(pad, dtype=jnp.int32) * 13) % N
    src3 = jnp.concatenate([src, fill]).reshape(NW, CH, C)
    dst3 = jnp.concatenate([dst, fill]).reshape(NW, CH, C)
    ew3 = jnp.concatenate([ew, jnp.zeros((pad,), jnp.float32)]
                          ).reshape(NW, CH, C)

    degp = _sc_degree(dst3, ew3)
    degT = degp[:, :N].T

    u0, dinv = _tc_first(degT, x, W0)
    sp0 = _sc_agg(u0, src3, dst3, ew3)
    u1 = _tc_mid(sp0, u0, dinv, b0.reshape(1, D), W1)
    sp1 = _sc_agg(u1, src3, dst3, ew3)
    u2 = _tc_mid(sp1, u1, dinv, b1.reshape(1, D), W2)
    sp2 = _sc_agg(u2, src3, dst3, ew3)
    out = _tc_head(sp2, u2, dinv, b2.reshape(1, D),
                   Wm1, bm1.reshape(1, -1), Wm2, bm2.reshape(1, -1))
    return out
